# baseline (speedup 1.0000x reference)
import functools

import jax
import jax.numpy as jnp
from jax import lax
from jax.experimental import pallas as pl
from jax.experimental.pallas import tpu as pltpu
from jax.experimental.pallas import tpu_sc as plsc

NUM_EMB = 64
EMB_DIM = 16
NC = 2
NS = 16
NW = NC * NS
LANES = 16
L_SEQ = 200
W = L_SEQ * EMB_DIM


def _make_expand(NR):
    rpw = NR // NW
    RS = 16
    nslab = rpw // RS
    SLAB_I = RS * L_SEQ
    NGRP = L_SEQ // LANES + 1
    mesh = plsc.VectorSubcoreMesh(core_axis_name="c", subcore_axis_name="s")

    @functools.partial(
        pl.kernel,
        out_type=jax.ShapeDtypeStruct((NR, W), jnp.float32),
        mesh=mesh,
        scratch_types=[
            pltpu.VMEM((NUM_EMB * EMB_DIM,), jnp.float32),
            pltpu.VMEM((2, EMB_DIM), jnp.float32),
            pltpu.VMEM((2, RS, L_SEQ), jnp.int32),
            pltpu.VMEM((2, RS, W), jnp.float32),
            pltpu.SemaphoreType.DMA,
            pltpu.SemaphoreType.DMA,
            pltpu.SemaphoreType.DMA,
            pltpu.SemaphoreType.DMA,
        ],
        compiler_params=pltpu.CompilerParams(
            use_tc_tiling_on_sc=True, needs_layout_passes=False),
    )
    def expand(tab_hbm, idx_hbm, gam_hbm, bet_hbm, out_hbm,
               tab_v, gb_v, idx_v, rows_v, i0, i1, s0, s1):
        isems = (i0, i1)
        ssems = (s0, s1)
        wid = lax.axis_index("s") * NC + lax.axis_index("c")
        base = wid * rpw
        pltpu.sync_copy(tab_hbm, tab_v)
        pltpu.sync_copy(gam_hbm, gb_v.at[0])
        pltpu.sync_copy(bet_hbm, gb_v.at[1])

        iota = lax.iota(jnp.int32, LANES)

        gam = gb_v[0, :]
        bet = gb_v[1, :]
        for r in range(NUM_EMB):
            row = tab_v[pl.ds(r * EMB_DIM, EMB_DIM)]
            m = jnp.sum(row) * (1.0 / EMB_DIM)
            c = row - m
            var = jnp.sum(c * c) * (1.0 / EMB_DIM) + 1e-5
            av = jnp.zeros((LANES,), jnp.float32) + var
            y = plsc.bitcast(
                jnp.int32(0x5F3759DF) - (plsc.bitcast(av, jnp.int32) >> 1),
                jnp.float32)
            for _ in range(3):
                y = y * (1.5 - 0.5 * av * y * y)
            tab_v[pl.ds(r * EMB_DIM, EMB_DIM)] = c * y * gam + bet

        def i_copy(i, b):
            return pltpu.make_async_copy(
                idx_hbm.at[pl.ds(base + i * RS, RS), :],
                idx_v.at[b], isems[b])

        def s_copy(i, b):
            return pltpu.make_async_copy(
                rows_v.at[b],
                out_hbm.at[pl.ds(base + i * RS, RS), :],
                ssems[b])

        def compute(b):
            def xrow(xr, carry):
                for g in range(L_SEQ // LANES):
                    l0 = g * LANES
                    v16 = idx_v[b, xr, pl.ds(l0, LANES)] * EMB_DIM
                    for j in range(LANES):
                        row = plsc.load_gather(tab_v, [iota + v16[j]])
                        rows_v[b, xr, pl.ds((l0 + j) * EMB_DIM, EMB_DIM)] = row
                v16t = idx_v[b, xr, pl.ds(L_SEQ - LANES, LANES)] * EMB_DIM
                for j in range(8, LANES):
                    row = plsc.load_gather(tab_v, [iota + v16t[j]])
                    rows_v[b, xr,
                           pl.ds((L_SEQ - LANES + j) * EMB_DIM, EMB_DIM)] = row
                return carry

            lax.fori_loop(0, RS, xrow, 0)

        i_copy(0, 0).start()
        i_copy(1, 1).start()

        def body(jp, carry):
            for b in range(2):
                i = jp * 2 + b
                i_copy(i, b).wait()

                @pl.when(jp >= 1)
                def _():
                    s_copy(i - 2, b).wait()

                compute(b)

                @pl.when(jp < nslab // 2 - 1)
                def _():
                    i_copy(i + 2, b).start()

                s_copy(i, b).start()
            return carry

        lax.fori_loop(0, nslab // 2, body, 0)
        for b in range(2):
            s_copy(nslab - 2 + b, b).wait()

    return expand


def kernel(x, table, gamma, beta):
    Bx, L = x.shape
    out = _make_expand(Bx)(table.reshape(-1), x, gamma, beta)
    return out.reshape(Bx, L, EMB_DIM)


# speedup vs baseline: 1.0040x; 1.0040x over previous
import functools

import jax
import jax.numpy as jnp
from jax import lax
from jax.experimental import pallas as pl
from jax.experimental.pallas import tpu as pltpu
from jax.experimental.pallas import tpu_sc as plsc

NUM_EMB = 64
EMB_DIM = 16
NC = 2
NS = 16
NW = NC * NS
LANES = 16
L_SEQ = 200
W = L_SEQ * EMB_DIM


def _ln_table_body(t_ref, g_ref, b_ref, o_ref):
    t = t_ref[...]
    m = jnp.mean(t, axis=1, keepdims=True)
    v = jnp.mean(jnp.square(t - m), axis=1, keepdims=True)
    o_ref[...] = (t - m) / jnp.sqrt(v + 1e-5) * g_ref[...] + b_ref[...]


def _ln_table(table, gamma, beta):
    return pl.pallas_call(
        _ln_table_body,
        out_shape=jax.ShapeDtypeStruct(table.shape, table.dtype),
    )(table, gamma, beta)


def _make_expand(NR):
    rpw = NR // NW
    RS = 16
    nslab = rpw // RS
    SLAB_I = RS * L_SEQ
    NGRP = L_SEQ // LANES + 1
    mesh = plsc.VectorSubcoreMesh(core_axis_name="c", subcore_axis_name="s")

    @functools.partial(
        pl.kernel,
        out_type=jax.ShapeDtypeStruct((NR, W), jnp.float32),
        mesh=mesh,
        scratch_types=[
            pltpu.VMEM((NUM_EMB * EMB_DIM,), jnp.float32),
            pltpu.VMEM((2, RS, L_SEQ), jnp.int32),
            pltpu.VMEM((2, RS, W), jnp.float32),
            pltpu.SemaphoreType.DMA,
            pltpu.SemaphoreType.DMA,
            pltpu.SemaphoreType.DMA,
            pltpu.SemaphoreType.DMA,
        ],
        compiler_params=pltpu.CompilerParams(
            use_tc_tiling_on_sc=True, needs_layout_passes=False),
    )
    def expand(tab_hbm, idx_hbm, out_hbm, tab_v, idx_v, rows_v,
               i0, i1, s0, s1):
        isems = (i0, i1)
        ssems = (s0, s1)
        wid = lax.axis_index("s") * NC + lax.axis_index("c")
        base = wid * rpw
        pltpu.sync_copy(tab_hbm, tab_v)

        iota = lax.iota(jnp.int32, LANES)

        def i_copy(i, b):
            return pltpu.make_async_copy(
                idx_hbm.at[pl.ds(base + i * RS, RS), :],
                idx_v.at[b], isems[b])

        def s_copy(i, b):
            return pltpu.make_async_copy(
                rows_v.at[b],
                out_hbm.at[pl.ds(base + i * RS, RS), :],
                ssems[b])

        def compute(b):
            def xrow(xr, carry):
                for g in range(L_SEQ // LANES):
                    l0 = g * LANES
                    v16 = idx_v[b, xr, pl.ds(l0, LANES)] * EMB_DIM
                    for j in range(LANES):
                        row = plsc.load_gather(tab_v, [iota + v16[j]])
                        rows_v[b, xr, pl.ds((l0 + j) * EMB_DIM, EMB_DIM)] = row
                v16t = idx_v[b, xr, pl.ds(L_SEQ - LANES, LANES)] * EMB_DIM
                for j in range(8, LANES):
                    row = plsc.load_gather(tab_v, [iota + v16t[j]])
                    rows_v[b, xr,
                           pl.ds((L_SEQ - LANES + j) * EMB_DIM, EMB_DIM)] = row
                return carry

            lax.fori_loop(0, RS, xrow, 0)

        i_copy(0, 0).start()
        i_copy(1, 1).start()

        def body(jp, carry):
            for b in range(2):
                i = jp * 2 + b
                i_copy(i, b).wait()

                @pl.when(jp >= 1)
                def _():
                    s_copy(i - 2, b).wait()

                compute(b)

                @pl.when(jp < nslab // 2 - 1)
                def _():
                    i_copy(i + 2, b).start()

                s_copy(i, b).start()
            return carry

        lax.fori_loop(0, nslab // 2, body, 0)
        for b in range(2):
            s_copy(nslab - 2 + b, b).wait()

    return expand


def kernel(x, table, gamma, beta):
    Bx, L = x.shape
    normed = _ln_table(table, gamma.reshape(1, EMB_DIM), beta.reshape(1, EMB_DIM))
    out = _make_expand(Bx)(normed.reshape(-1), x)
    return out.reshape(Bx, L, EMB_DIM)
